# Initial kernel scaffold; baseline (speedup 1.0000x reference)
import jax
import jax.numpy as jnp
from jax.experimental import pallas as pl


def kernel(token_ids, weights):
    raise NotImplementedError("write your pallas kernel here")


# speedup vs baseline: 4.5361x; 4.5361x over previous
import functools

import jax
import jax.numpy as jnp
from jax import lax
from jax.experimental import pallas as pl
from jax.experimental.pallas import tpu as pltpu
from jax.experimental.pallas import tpu_sc as plsc

_CH = 128


@functools.cache
def _build(num_chunks: int, dim: int):
    info = plsc.get_sparse_core_info()
    nc, ns = info.num_cores, info.num_subcores
    nw = nc * ns
    b_per_w = num_chunks * _CH
    mesh = plsc.VectorSubcoreMesh(core_axis_name="c", subcore_axis_name="s")

    @functools.partial(
        pl.kernel,
        mesh=mesh,
        compiler_params=pltpu.CompilerParams(use_tc_tiling_on_sc=False),
        out_type=jax.ShapeDtypeStruct((nw * b_per_w, dim), jnp.float32),
        scratch_types=[
            pltpu.VMEM((num_chunks, _CH), jnp.int32),
            pltpu.VMEM((2, _CH, dim), jnp.float32),
            pltpu.SemaphoreType.DMA,
            pltpu.SemaphoreType.DMA,
            pltpu.SemaphoreType.DMA,
            pltpu.SemaphoreType.DMA,
        ],
    )
    def k(idx_hbm, table_hbm, out_hbm, idx_v, bufs, g0, g1, w0, w1):
        gsems = (g0, g1)
        wsems = (w0, w1)
        wid = lax.axis_index("s") * nc + lax.axis_index("c")
        base = wid * b_per_w
        pltpu.sync_copy(idx_hbm.at[wid], idx_v)

        pltpu.async_copy(table_hbm.at[idx_v.at[0]], bufs.at[0], gsems[0])

        def body(g, _):
            for b in range(2):
                c = 2 * g + b
                nxt = c + 1

                @pl.when(nxt < num_chunks)
                def _():
                    @pl.when(c >= 1)
                    def _():
                        pltpu.make_async_copy(
                            bufs.at[1 - b], out_hbm.at[pl.ds(0, _CH)], wsems[1 - b]
                        ).wait()

                    pltpu.async_copy(
                        table_hbm.at[idx_v.at[nxt]], bufs.at[1 - b], gsems[1 - b]
                    )

                pltpu.make_async_copy(
                    table_hbm.at[idx_v.at[c]], bufs.at[b], gsems[b]
                ).wait()
                pltpu.async_copy(
                    bufs.at[b], out_hbm.at[pl.ds(base + c * _CH, _CH)], wsems[b]
                )
            return 0

        lax.fori_loop(0, num_chunks // 2, body, 0, unroll=False)
        for b in range(2):
            pltpu.make_async_copy(
                bufs.at[b], out_hbm.at[pl.ds(0, _CH)], wsems[b]
            ).wait()

    return k


def kernel(token_ids, weights):
    bsz, seq = token_ids.shape
    dim = weights.shape[1]
    total = bsz * seq
    info = plsc.get_sparse_core_info()
    nw = info.num_cores * info.num_subcores
    num_chunks = total // (nw * _CH)
    k = _build(num_chunks, dim)
    idx = token_ids.reshape(nw, num_chunks, _CH).astype(jnp.int32)
    out = k(idx, weights)
    return out.reshape(bsz, seq, dim)


# speedup vs baseline: 4.6643x; 1.0283x over previous
import functools

import jax
import jax.numpy as jnp
from jax import lax
from jax.experimental import pallas as pl
from jax.experimental.pallas import tpu as pltpu
from jax.experimental.pallas import tpu_sc as plsc

_CH = 800
_NBUF = 2


@functools.cache
def _build(b_per_w: int, dim: int):
    info = plsc.get_sparse_core_info()
    nc, ns = info.num_cores, info.num_subcores
    nw = nc * ns
    n_chunks = b_per_w // _CH
    pf = _NBUF - 1
    assert n_chunks % _NBUF == 0 and n_chunks > pf
    mesh = plsc.VectorSubcoreMesh(core_axis_name="c", subcore_axis_name="s")

    @functools.partial(
        pl.kernel,
        mesh=mesh,
        compiler_params=pltpu.CompilerParams(use_tc_tiling_on_sc=False),
        out_type=jax.ShapeDtypeStruct((nw * b_per_w, dim), jnp.float32),
        scratch_types=[
            pltpu.VMEM((b_per_w,), jnp.int32),
            pltpu.VMEM((_NBUF, _CH, dim), jnp.float32),
            [pltpu.SemaphoreType.DMA] * _NBUF,
            [pltpu.SemaphoreType.DMA] * _NBUF,
        ],
    )
    def k(idx_hbm, table_hbm, out_hbm, idx_v, bufs, gsems, wsems):
        wid = lax.axis_index("s") * nc + lax.axis_index("c")
        base = wid * b_per_w
        pltpu.sync_copy(idx_hbm.at[wid], idx_v)

        def gather(c, p):
            pltpu.async_copy(
                table_hbm.at[idx_v.at[pl.ds(c * _CH, _CH)]], bufs.at[p], gsems[p]
            )

        for j in range(pf):
            gather(j, j)

        def body(g, _):
            for b in range(_NBUF):
                c = g * _NBUF + b
                fut = c + pf

                @pl.when(fut < n_chunks)
                def _():
                    fp = (b + pf) % _NBUF

                    @pl.when(c >= 1)
                    def _():
                        pltpu.make_async_copy(
                            bufs.at[fp], out_hbm.at[pl.ds(0, _CH)], wsems[fp]
                        ).wait()

                    gather(fut, fp)

                pltpu.make_async_copy(
                    table_hbm.at[idx_v.at[pl.ds(c * _CH, _CH)]], bufs.at[b], gsems[b]
                ).wait()
                pltpu.async_copy(
                    bufs.at[b], out_hbm.at[pl.ds(base + c * _CH, _CH)], wsems[b]
                )
            return 0

        lax.fori_loop(0, n_chunks // _NBUF, body, 0, unroll=False)
        for b in range(_NBUF):
            pltpu.make_async_copy(
                bufs.at[b], out_hbm.at[pl.ds(0, _CH)], wsems[b]
            ).wait()

    return k


def kernel(token_ids, weights):
    bsz, seq = token_ids.shape
    dim = weights.shape[1]
    total = bsz * seq
    info = plsc.get_sparse_core_info()
    nw = info.num_cores * info.num_subcores
    b_per_w = total // nw
    k = _build(b_per_w, dim)
    idx = token_ids.reshape(nw, b_per_w).astype(jnp.int32)
    out = k(idx, weights)
    return out.reshape(bsz, seq, dim)


# speedup vs baseline: 4.6730x; 1.0019x over previous
import functools

import jax
import jax.numpy as jnp
from jax import lax
from jax.experimental import pallas as pl
from jax.experimental.pallas import tpu as pltpu
from jax.experimental.pallas import tpu_sc as plsc

_CH = 400
_NBUF = 4


@functools.cache
def _build(b_per_w: int, dim: int):
    info = plsc.get_sparse_core_info()
    nc, ns = info.num_cores, info.num_subcores
    nw = nc * ns
    n_chunks = b_per_w // _CH
    pf = _NBUF - 1
    assert n_chunks % _NBUF == 0 and n_chunks > pf
    mesh = plsc.VectorSubcoreMesh(core_axis_name="c", subcore_axis_name="s")

    @functools.partial(
        pl.kernel,
        mesh=mesh,
        compiler_params=pltpu.CompilerParams(use_tc_tiling_on_sc=False),
        out_type=jax.ShapeDtypeStruct((nw * b_per_w, dim), jnp.float32),
        scratch_types=[
            pltpu.VMEM((b_per_w,), jnp.int32),
            pltpu.VMEM((_NBUF, _CH, dim), jnp.float32),
            [pltpu.SemaphoreType.DMA] * _NBUF,
            [pltpu.SemaphoreType.DMA] * _NBUF,
        ],
    )
    def k(idx_hbm, table_hbm, out_hbm, idx_v, bufs, gsems, wsems):
        wid = lax.axis_index("s") * nc + lax.axis_index("c")
        base = wid * b_per_w
        pltpu.sync_copy(idx_hbm.at[wid], idx_v)

        def gather(c, p):
            pltpu.async_copy(
                table_hbm.at[idx_v.at[pl.ds(c * _CH, _CH)]], bufs.at[p], gsems[p]
            )

        for j in range(pf):
            gather(j, j)

        def body(g, _):
            for b in range(_NBUF):
                c = g * _NBUF + b
                fut = c + pf

                @pl.when(fut < n_chunks)
                def _():
                    fp = (b + pf) % _NBUF

                    @pl.when(c >= 1)
                    def _():
                        pltpu.make_async_copy(
                            bufs.at[fp], out_hbm.at[pl.ds(0, _CH)], wsems[fp]
                        ).wait()

                    gather(fut, fp)

                pltpu.make_async_copy(
                    table_hbm.at[idx_v.at[pl.ds(c * _CH, _CH)]], bufs.at[b], gsems[b]
                ).wait()
                pltpu.async_copy(
                    bufs.at[b], out_hbm.at[pl.ds(base + c * _CH, _CH)], wsems[b]
                )
            return 0

        lax.fori_loop(0, n_chunks // _NBUF, body, 0, unroll=False)
        for b in range(_NBUF):
            pltpu.make_async_copy(
                bufs.at[b], out_hbm.at[pl.ds(0, _CH)], wsems[b]
            ).wait()

    return k


def kernel(token_ids, weights):
    bsz, seq = token_ids.shape
    dim = weights.shape[1]
    total = bsz * seq
    info = plsc.get_sparse_core_info()
    nw = info.num_cores * info.num_subcores
    b_per_w = total // nw
    k = _build(b_per_w, dim)
    idx = token_ids.reshape(nw, b_per_w).astype(jnp.int32)
    out = k(idx, weights)
    return out.reshape(bsz, seq, dim)
